# baseline (speedup 1.0000x reference)
import jax
import jax.numpy as jnp
from jax.experimental import pallas as pl
from jax.experimental.pallas import tpu as pltpu

_B, _H, _W, _C = 16, 32, 32, 768
_ROWS = _H * _W
_BR = 512
_BBG = 8
_NRC = _ROWS // _BR
_NBG = _B // _BBG


def _body(table_ref, mid_ref, x_ref, scale_ref, mean_ref, out_ref, q_ref):
    @pl.when(pl.program_id(1) == 0)
    def _compute_q():
        s = jnp.abs(scale_ref[...])
        q = jnp.full(s.shape, table_ref[0], dtype=jnp.float32)
        for j in range(mid_ref.shape[0]):
            q = jnp.where(s > mid_ref[j], table_ref[j + 1], q)
        q_ref[...] = q

    q = q_ref[...][None, :, :]
    m = mean_ref[...][None, :, :]
    x = x_ref[...]
    out_ref[...] = jnp.round((x - m) / q) * q + m


def kernel(inputs, scale, mean, scale_table, midpoints):
    x = inputs.reshape(_B, _ROWS, _C)
    s = scale.reshape(_ROWS, _C)
    m = mean.reshape(_ROWS, _C)

    out = pl.pallas_call(
        _body,
        grid=(_NRC, _NBG),
        in_specs=[
            pl.BlockSpec(memory_space=pltpu.SMEM),
            pl.BlockSpec(memory_space=pltpu.SMEM),
            pl.BlockSpec((_BBG, _BR, _C), lambda i, j: (j, i, 0)),
            pl.BlockSpec((_BR, _C), lambda i, j: (i, 0)),
            pl.BlockSpec((_BR, _C), lambda i, j: (i, 0)),
        ],
        out_specs=pl.BlockSpec((_BBG, _BR, _C), lambda i, j: (j, i, 0)),
        out_shape=jax.ShapeDtypeStruct((_B, _ROWS, _C), jnp.float32),
        scratch_shapes=[
            pltpu.VMEM((_BR, _C), jnp.float32),
        ],
        compiler_params=pltpu.CompilerParams(
            dimension_semantics=("arbitrary", "arbitrary"),
        ),
    )(scale_table, midpoints, x, s, m)
    return out.reshape(_B, _H, _W, _C)


# speedup vs baseline: 1.1135x; 1.1135x over previous
import jax
import jax.numpy as jnp
from jax.experimental import pallas as pl
from jax.experimental.pallas import tpu as pltpu

_B, _H, _W, _C = 16, 32, 32, 768
_ROWS = _H * _W
_BR = 128


def _body(table_ref, mid_ref, x_ref, scale_ref, mean_ref, out_ref):
    s = jnp.abs(scale_ref[...])
    q = jnp.full(s.shape, table_ref[0], dtype=jnp.float32)
    for j in range(mid_ref.shape[0]):
        q = jnp.where(s > mid_ref[j], table_ref[j + 1], q)
    m = mean_ref[...]
    x = x_ref[...]
    qb = q[None, :, :]
    mb = m[None, :, :]
    out_ref[...] = jnp.round((x - mb) / qb) * qb + mb


def kernel(inputs, scale, mean, scale_table, midpoints):
    x = inputs.reshape(_B, _ROWS, _C)
    s = scale.reshape(_ROWS, _C)
    m = mean.reshape(_ROWS, _C)

    grid = (_ROWS // _BR,)
    out = pl.pallas_call(
        _body,
        grid=grid,
        in_specs=[
            pl.BlockSpec(memory_space=pltpu.SMEM),
            pl.BlockSpec(memory_space=pltpu.SMEM),
            pl.BlockSpec((_B, _BR, _C), lambda i: (0, i, 0)),
            pl.BlockSpec((_BR, _C), lambda i: (i, 0)),
            pl.BlockSpec((_BR, _C), lambda i: (i, 0)),
        ],
        out_specs=pl.BlockSpec((_B, _BR, _C), lambda i: (0, i, 0)),
        out_shape=jax.ShapeDtypeStruct((_B, _ROWS, _C), jnp.float32),
        compiler_params=pltpu.CompilerParams(
            dimension_semantics=("parallel",),
        ),
    )(scale_table, midpoints, x, s, m)
    return out.reshape(_B, _H, _W, _C)


# speedup vs baseline: 1.1139x; 1.0004x over previous
import jax
import jax.numpy as jnp
from jax.experimental import pallas as pl
from jax.experimental.pallas import tpu as pltpu

_B, _H, _W, _C = 16, 32, 32, 768
_ROWS = _H * _W
_BR = 128


def _body(table_ref, mid_ref, x_ref, scale_ref, mean_ref, out_ref):
    s = jnp.abs(scale_ref[...])
    q = jnp.full(s.shape, table_ref[0], dtype=jnp.float32)
    for j in range(mid_ref.shape[0]):
        q = jnp.where(s > mid_ref[j], table_ref[j + 1], q)
    m = mean_ref[...]
    x = x_ref[...]
    qb = q[None, :, :]
    mb = m[None, :, :]
    out_ref[...] = jnp.round((x - mb) / qb) * qb + mb


def kernel(inputs, scale, mean, scale_table, midpoints):
    x = inputs.reshape(_B, _ROWS, _C)
    s = scale.reshape(_ROWS, _C)
    m = mean.reshape(_ROWS, _C)

    grid = (_ROWS // _BR,)
    out = pl.pallas_call(
        _body,
        grid=grid,
        in_specs=[
            pl.BlockSpec(memory_space=pltpu.SMEM),
            pl.BlockSpec(memory_space=pltpu.SMEM),
            pl.BlockSpec((_B, _BR, _C), lambda i: (0, i, 0)),
            pl.BlockSpec((_BR, _C), lambda i: (i, 0)),
            pl.BlockSpec((_BR, _C), lambda i: (i, 0)),
        ],
        out_specs=pl.BlockSpec((_B, _BR, _C), lambda i: (0, i, 0)),
        out_shape=jax.ShapeDtypeStruct((_B, _ROWS, _C), jnp.float32),
        compiler_params=pltpu.CompilerParams(
            dimension_semantics=("arbitrary",),
        ),
    )(scale_table, midpoints, x, s, m)
    return out.reshape(_B, _H, _W, _C)
